# baseline (speedup 1.0000x reference)
import functools

import jax
import jax.numpy as jnp
from jax import lax
from jax.experimental import pallas as pl
from jax.experimental.pallas import tpu as pltpu
from jax.experimental.pallas import tpu_sc as plsc

F32 = jnp.float32
_NC = 2
_NS = 16
_L = 16
_NW = _NC * _NS

_HEADS = 4
_DH = 32
_HID = _HEADS * _DH
_REC = 144
_CHUNK = 48

_SQRT_HALF = 0.7071067811865476


def _gelu(x):
    return 0.5 * x * (1.0 + lax.erf(x * _SQRT_HALF))


def _layer_norm(x, w, b):
    mu = jnp.mean(x, axis=-1, keepdims=True)
    var = jnp.mean((x - mu) ** 2, axis=-1, keepdims=True)
    return (x - mu) * lax.rsqrt(var + 1e-5) * w + b


def _attn_cols(h, att_flat):
    hw = h * att_flat
    cols = [jnp.sum(hw[:, _DH * k:_DH * (k + 1)], axis=1, keepdims=True)
            for k in range(_HEADS)]
    return jnp.concatenate(cols, axis=1)


def _combine(p0, p1, bias):
    acc = p0[:, :_HID] + p1[:, :_HID]
    den = p0[:, _HID:_HID + _HEADS] + p1[:, _HID:_HID + _HEADS]
    outs = []
    for k in range(_HEADS):
        dk = den[:, k:k + 1] + 1e-16
        outs.append(acc[:, _DH * k:_DH * (k + 1)] / dk)
    return jnp.concatenate(outs, axis=1) + bias


def _tables(h, a_src_flat, a_dst_flat):
    bn = h.shape[0]
    zsrc = jnp.zeros((bn, _REC - _HID - _HEADS), F32)
    zdst = jnp.zeros((bn, _L - _HEADS), F32)
    tsrc = jnp.concatenate([h, _attn_cols(h, a_src_flat), zsrc], axis=1)
    tdst = jnp.concatenate([_attn_cols(h, a_dst_flat), zdst], axis=1)
    return tsrc, tdst


def _s1_body(x_ref, win_ref, bin_ref, lnw_ref, lnb_ref, w1_ref, as_ref,
             ad_ref, tsrc_ref, tdst_ref):
    x = x_ref[...]
    y = jnp.dot(x, win_ref[...], preferred_element_type=F32) + bin_ref[...]
    y = _gelu(_layer_norm(y, lnw_ref[...], lnb_ref[...]))
    h = jnp.dot(y, w1_ref[...], preferred_element_type=F32)
    tsrc, tdst = _tables(h, as_ref[...], ad_ref[...])
    tsrc_ref[...] = tsrc
    tdst_ref[...] = tdst


def _s2_body(p0_ref, p1_ref, b1_ref, lnw_ref, lnb_ref, w2_ref, as_ref,
             ad_ref, tsrc_ref, tdst_ref):
    x = _combine(p0_ref[...], p1_ref[...], b1_ref[...])
    y = _gelu(_layer_norm(x, lnw_ref[...], lnb_ref[...]))
    h = jnp.dot(y, w2_ref[...], preferred_element_type=F32)
    tsrc, tdst = _tables(h, as_ref[...], ad_ref[...])
    tsrc_ref[...] = tsrc
    tdst_ref[...] = tdst


def _s3_body(p0_ref, p1_ref, b2_ref, wout_ref, bout_ref, out_ref):
    x = _combine(p0_ref[...], p1_ref[...], b2_ref[...])
    out_ref[...] = jnp.dot(x, wout_ref[...], preferred_element_type=F32) \
        + bout_ref[...]


def _row_spec(bn, cols):
    return pl.BlockSpec((bn, cols), lambda i: (i, 0))


def _full_spec(shape):
    return pl.BlockSpec(shape, lambda i: (0,) * len(shape))


def _stage1(x, w_in, b_in, ln_w, ln_b, w1, a_s, a_d, bn):
    n, in_ch = x.shape
    grid = (n // bn,)
    return pl.pallas_call(
        _s1_body,
        grid=grid,
        in_specs=[
            _row_spec(bn, in_ch),
            _full_spec(w_in.shape), _full_spec(b_in.shape),
            _full_spec(ln_w.shape), _full_spec(ln_b.shape),
            _full_spec(w1.shape), _full_spec(a_s.shape),
            _full_spec(a_d.shape),
        ],
        out_specs=[_row_spec(bn, _REC), _row_spec(bn, _L)],
        out_shape=[
            jax.ShapeDtypeStruct((n, _REC), F32),
            jax.ShapeDtypeStruct((n, _L), F32),
        ],
    )(x, w_in, b_in, ln_w, ln_b, w1, a_s, a_d)


def _stage2(p0, p1, b1, ln_w, ln_b, w2, a_s, a_d, bn):
    n = p0.shape[0]
    grid = (n // bn,)
    return pl.pallas_call(
        _s2_body,
        grid=grid,
        in_specs=[
            _row_spec(bn, _REC), _row_spec(bn, _REC),
            _full_spec(b1.shape),
            _full_spec(ln_w.shape), _full_spec(ln_b.shape),
            _full_spec(w2.shape), _full_spec(a_s.shape),
            _full_spec(a_d.shape),
        ],
        out_specs=[_row_spec(bn, _REC), _row_spec(bn, _L)],
        out_shape=[
            jax.ShapeDtypeStruct((n, _REC), F32),
            jax.ShapeDtypeStruct((n, _L), F32),
        ],
    )(p0, p1, b1, ln_w, ln_b, w2, a_s, a_d)


def _stage3(p0, p1, b2, w_out, b_out, bn):
    n = p0.shape[0]
    out_ch = w_out.shape[1]
    grid = (n // bn,)
    return pl.pallas_call(
        _s3_body,
        grid=grid,
        in_specs=[
            _row_spec(bn, _REC), _row_spec(bn, _REC),
            _full_spec(b2.shape),
            _full_spec(w_out.shape), _full_spec(b_out.shape),
        ],
        out_specs=_row_spec(bn, out_ch),
        out_shape=jax.ShapeDtypeStruct((n, out_ch), F32),
    )(p0, p1, b2, w_out, b_out)


def _edge_npad(n):
    gran = _NS * 8
    return ((n + gran - 1) // gran) * gran


_NBUF = 2


@functools.lru_cache(maxsize=None)
def _edge_pass(n, nchunk):
    assert nchunk % _NBUF == 0
    npad = _edge_npad(n)
    rpt = npad // _NS
    zsizes = [_CHUNK] * (rpt // _CHUNK)
    if rpt % _CHUNK:
        zsizes.append(rpt % _CHUNK)

    mesh = plsc.VectorSubcoreMesh(core_axis_name="c", subcore_axis_name="s",
                                  num_cores=_NC, num_subcores=_NS)

    scratch = [
        pltpu.VMEM_SHARED((npad, _REC), F32),
        pltpu.VMEM((nchunk, _CHUNK), jnp.int32),
        pltpu.VMEM((nchunk, _CHUNK), jnp.int32),
    ]
    scratch += [pltpu.VMEM((_CHUNK, _REC), F32) for _ in range(_NBUF)]
    scratch += [pltpu.VMEM((_CHUNK, _L), F32) for _ in range(_NBUF)]
    scratch += [pltpu.SemaphoreType.DMA for _ in range(3 * _NBUF)]

    @functools.partial(
        pl.kernel,
        out_type=[jax.ShapeDtypeStruct((npad, _REC), F32),
                  jax.ShapeDtypeStruct((npad, _REC), F32)],
        mesh=mesh,
        compiler_params=pltpu.CompilerParams(use_tc_tiling_on_sc=False),
        scratch_types=scratch,
    )
    def edge_kernel(tsrc, tdst, srcs, dsts, out0, out1, acc_sh, ixs, ixd,
                    *bufs):
        rv = bufs[0:_NBUF]
        dv = bufs[_NBUF:2 * _NBUF]
        semr = bufs[2 * _NBUF:3 * _NBUF]
        semd = bufs[3 * _NBUF:4 * _NBUF]
        sems = bufs[4 * _NBUF:5 * _NBUF]
        cid = lax.axis_index("c")
        sid = lax.axis_index("s")
        wid = sid * _NC + cid
        lanes = lax.iota(jnp.int32, _L)
        head_mask = lanes < _HEADS

        pltpu.sync_copy(srcs.at[pl.ds(wid * nchunk, nchunk)], ixs)
        pltpu.sync_copy(dsts.at[pl.ds(wid * nchunk, nchunk)], ixd)

        zero = jnp.zeros((_L,), F32)

        def zrow(i, _):
            for j in range(_REC // _L):
                rv[0][i, pl.ds(j * _L, _L)] = zero
            return 0

        lax.fori_loop(0, _CHUNK, zrow, 0)
        for k, zs in enumerate(zsizes):
            pltpu.sync_copy(rv[0].at[pl.ds(0, zs)],
                            acc_sh.at[pl.ds(sid * rpt + k * _CHUNK, zs)])
        plsc.subcore_barrier()

        def start_gather(c, b):
            pltpu.async_copy(tsrc.at[ixs.at[c]], rv[b], semr[b])
            pltpu.async_copy(tdst.at[ixd.at[c]], dv[b], semd[b])

        def wait_gather(c, b):
            pltpu.make_async_copy(tsrc.at[ixs.at[c]], rv[b], semr[b]).wait()
            pltpu.make_async_copy(tdst.at[ixd.at[c]], dv[b], semd[b]).wait()

        def start_scatter(c, b):
            pltpu.async_copy(rv[b], acc_sh.at[ixd.at[c]], sems[b], add=True)

        def wait_scatter(c, b):
            del c
            pltpu.make_async_copy(tsrc.at[pl.ds(0, _CHUNK)], rv[b],
                                  sems[b]).wait()

        def compute(b):
            rbuf = rv[b]
            dbuf = dv[b]

            def edge(i, _):
                for u in range(8):
                    e = 8 * i + u
                    asv = rbuf[e, pl.ds(_HID, _L)]
                    adv = dbuf[e, pl.ds(0, _L)]
                    t = asv + adv
                    t = jnp.maximum(t, t * 0.2)
                    s = jnp.exp(t)
                    rbuf[e, pl.ds(_HID, _L)] = jnp.where(head_mask, s, 0.0)
                    for k in range(_HEADS):
                        sk = jnp.broadcast_to(s[k], (_L,))
                        c0 = k * _DH
                        rbuf[e, pl.ds(c0, _L)] = rbuf[e, pl.ds(c0, _L)] * sk
                        rbuf[e, pl.ds(c0 + _L, _L)] = \
                            rbuf[e, pl.ds(c0 + _L, _L)] * sk
                return 0

            lax.fori_loop(0, _CHUNK // 8, edge, 0)

        start_gather(0, 0)

        def outer(i, _):
            c0 = i * _NBUF
            for b in range(_NBUF):
                c = c0 + b
                o = 1 - b

                wait_gather(c, b)

                @pl.when(c >= 1)
                def _():
                    wait_scatter(c - 1, o)

                @pl.when(c + 1 < nchunk)
                def _():
                    start_gather(c + 1, o)

                compute(b)
                start_scatter(c, b)
            return 0

        lax.fori_loop(0, nchunk // _NBUF, outer, 0)
        wait_scatter(nchunk - 1, (nchunk - 1) % _NBUF)

        plsc.subcore_barrier()

        @pl.when(cid == 0)
        def _():
            pltpu.sync_copy(acc_sh.at[pl.ds(sid * rpt, rpt)],
                            out0.at[pl.ds(sid * rpt, rpt)])

        @pl.when(cid == 1)
        def _():
            pltpu.sync_copy(acc_sh.at[pl.ds(sid * rpt, rpt)],
                            out1.at[pl.ds(sid * rpt, rpt)])

    return edge_kernel


def kernel(x, edge_index, W_in, b_in, ln1_w, ln1_b, W1, att_src1, att_dst1,
           b1, ln2_w, ln2_b, W2, att_src2, att_dst2, b2, W_out, b_out):
    n = x.shape[0]
    e = edge_index.shape[1]
    r2 = lambda v: v.reshape(1, -1)
    npad = _edge_npad(n)
    bn = npad // 8

    group = _NW * _CHUNK * _NBUF
    e_pad = ((e + group - 1) // group) * group
    nchunk = e_pad // (_NW * _CHUNK)
    pad = jnp.full((2, e_pad - e), n, jnp.int32)
    ei = jnp.concatenate([edge_index, pad], axis=1)
    src = ei[0].reshape(_NW * nchunk, _CHUNK)
    dst = ei[1].reshape(_NW * nchunk, _CHUNK)
    x_p = jnp.concatenate([x, jnp.zeros((npad - n, x.shape[1]), F32)])
    ep = _edge_pass(n, nchunk)

    tsrc1, tdst1 = _stage1(x_p, W_in, r2(b_in), r2(ln1_w), r2(ln1_b), W1,
                           r2(att_src1), r2(att_dst1), bn)
    p0, p1 = ep(tsrc1, tdst1, src, dst)
    tsrc2, tdst2 = _stage2(p0, p1, r2(b1), r2(ln2_w),
                           r2(ln2_b), W2, r2(att_src2), r2(att_dst2), bn)
    q0, q1 = ep(tsrc2, tdst2, src, dst)
    out = _stage3(q0, q1, r2(b2), W_out, b_out[None, :], bn)
    return out[:n]


# speedup vs baseline: 1.3941x; 1.3941x over previous
import functools

import jax
import jax.numpy as jnp
from jax import lax
from jax.experimental import pallas as pl
from jax.experimental.pallas import tpu as pltpu
from jax.experimental.pallas import tpu_sc as plsc

F32 = jnp.float32
_NC = 2
_NS = 16
_L = 16
_NW = _NC * _NS

_HEADS = 4
_DH = 32
_HID = _HEADS * _DH
_REC = 144
_CHUNK = 48

_SQRT_HALF = 0.7071067811865476


def _gelu(x):
    return 0.5 * x * (1.0 + lax.erf(x * _SQRT_HALF))


def _layer_norm(x, w, b):
    mu = jnp.mean(x, axis=-1, keepdims=True)
    var = jnp.mean((x - mu) ** 2, axis=-1, keepdims=True)
    return (x - mu) * lax.rsqrt(var + 1e-5) * w + b


def _attn_cols(h, att_flat):
    hw = h * att_flat
    cols = [jnp.sum(hw[:, _DH * k:_DH * (k + 1)], axis=1, keepdims=True)
            for k in range(_HEADS)]
    return jnp.concatenate(cols, axis=1)


def _combine(p0, p1, bias):
    acc = p0[:, :_HID] + p1[:, :_HID]
    den = p0[:, _HID:_HID + _HEADS] + p1[:, _HID:_HID + _HEADS]
    outs = []
    for k in range(_HEADS):
        dk = den[:, k:k + 1] + 1e-16
        outs.append(acc[:, _DH * k:_DH * (k + 1)] / dk)
    return jnp.concatenate(outs, axis=1) + bias


def _tables(h, a_src_flat, a_dst_flat):
    bn = h.shape[0]
    zsrc = jnp.zeros((bn, _REC - _HID - _HEADS), F32)
    zdst = jnp.zeros((bn, _L - _HEADS), F32)
    tsrc = jnp.concatenate([h, _attn_cols(h, a_src_flat), zsrc], axis=1)
    tdst = jnp.concatenate([_attn_cols(h, a_dst_flat), zdst], axis=1)
    return tsrc, tdst


def _s1_body(x_ref, win_ref, bin_ref, lnw_ref, lnb_ref, w1_ref, as_ref,
             ad_ref, tsrc_ref, tdst_ref):
    x = x_ref[...]
    y = jnp.dot(x, win_ref[...], preferred_element_type=F32) + bin_ref[...]
    y = _gelu(_layer_norm(y, lnw_ref[...], lnb_ref[...]))
    h = jnp.dot(y, w1_ref[...], preferred_element_type=F32)
    tsrc, tdst = _tables(h, as_ref[...], ad_ref[...])
    tsrc_ref[...] = tsrc
    tdst_ref[...] = tdst


def _s2_body(p0_ref, p1_ref, b1_ref, lnw_ref, lnb_ref, w2_ref, as_ref,
             ad_ref, tsrc_ref, tdst_ref):
    x = _combine(p0_ref[...], p1_ref[...], b1_ref[...])
    y = _gelu(_layer_norm(x, lnw_ref[...], lnb_ref[...]))
    h = jnp.dot(y, w2_ref[...], preferred_element_type=F32)
    tsrc, tdst = _tables(h, as_ref[...], ad_ref[...])
    tsrc_ref[...] = tsrc
    tdst_ref[...] = tdst


def _s3_body(p0_ref, p1_ref, b2_ref, wout_ref, bout_ref, out_ref):
    x = _combine(p0_ref[...], p1_ref[...], b2_ref[...])
    out_ref[...] = jnp.dot(x, wout_ref[...], preferred_element_type=F32) \
        + bout_ref[...]


def _row_spec(bn, cols):
    return pl.BlockSpec((bn, cols), lambda i: (i, 0))


def _full_spec(shape):
    return pl.BlockSpec(shape, lambda i: (0,) * len(shape))


def _stage1(x, w_in, b_in, ln_w, ln_b, w1, a_s, a_d, bn):
    n, in_ch = x.shape
    grid = (n // bn,)
    return pl.pallas_call(
        _s1_body,
        grid=grid,
        in_specs=[
            _row_spec(bn, in_ch),
            _full_spec(w_in.shape), _full_spec(b_in.shape),
            _full_spec(ln_w.shape), _full_spec(ln_b.shape),
            _full_spec(w1.shape), _full_spec(a_s.shape),
            _full_spec(a_d.shape),
        ],
        out_specs=[_row_spec(bn, _REC), _row_spec(bn, _L)],
        out_shape=[
            jax.ShapeDtypeStruct((n, _REC), F32),
            jax.ShapeDtypeStruct((n, _L), F32),
        ],
    )(x, w_in, b_in, ln_w, ln_b, w1, a_s, a_d)


def _stage2(p0, p1, b1, ln_w, ln_b, w2, a_s, a_d, bn):
    n = p0.shape[0]
    grid = (n // bn,)
    return pl.pallas_call(
        _s2_body,
        grid=grid,
        in_specs=[
            _row_spec(bn, _REC), _row_spec(bn, _REC),
            _full_spec(b1.shape),
            _full_spec(ln_w.shape), _full_spec(ln_b.shape),
            _full_spec(w2.shape), _full_spec(a_s.shape),
            _full_spec(a_d.shape),
        ],
        out_specs=[_row_spec(bn, _REC), _row_spec(bn, _L)],
        out_shape=[
            jax.ShapeDtypeStruct((n, _REC), F32),
            jax.ShapeDtypeStruct((n, _L), F32),
        ],
    )(p0, p1, b1, ln_w, ln_b, w2, a_s, a_d)


def _stage3(p0, p1, b2, w_out, b_out, bn):
    n = p0.shape[0]
    out_ch = w_out.shape[1]
    grid = (n // bn,)
    return pl.pallas_call(
        _s3_body,
        grid=grid,
        in_specs=[
            _row_spec(bn, _REC), _row_spec(bn, _REC),
            _full_spec(b2.shape),
            _full_spec(w_out.shape), _full_spec(b_out.shape),
        ],
        out_specs=_row_spec(bn, out_ch),
        out_shape=jax.ShapeDtypeStruct((n, out_ch), F32),
    )(p0, p1, b2, w_out, b_out)


def _edge_npad(n):
    gran = _NS * 8
    return ((n + gran - 1) // gran) * gran


_NBUF = 2


@functools.lru_cache(maxsize=None)
def _edge_pass(n, nchunk):
    assert nchunk % _NBUF == 0
    npad = _edge_npad(n)
    rpt = npad // _NS
    zsizes = [_CHUNK] * (rpt // _CHUNK)
    if rpt % _CHUNK:
        zsizes.append(rpt % _CHUNK)

    mesh = plsc.VectorSubcoreMesh(core_axis_name="c", subcore_axis_name="s",
                                  num_cores=_NC, num_subcores=_NS)

    scratch = [
        pltpu.VMEM_SHARED((npad, _REC), F32),
        pltpu.VMEM((nchunk, _CHUNK), jnp.int32),
        pltpu.VMEM((nchunk, _CHUNK), jnp.int32),
    ]
    scratch += [pltpu.VMEM((_CHUNK, _REC), F32) for _ in range(_NBUF)]
    scratch += [pltpu.VMEM((_CHUNK, _L), F32) for _ in range(_NBUF)]
    scratch += [pltpu.SemaphoreType.DMA for _ in range(3 * _NBUF)]

    @functools.partial(
        pl.kernel,
        out_type=[jax.ShapeDtypeStruct((npad, _REC), F32),
                  jax.ShapeDtypeStruct((npad, _REC), F32)],
        mesh=mesh,
        compiler_params=pltpu.CompilerParams(use_tc_tiling_on_sc=False),
        scratch_types=scratch,
    )
    def edge_kernel(tsrc, tdst, srcs, dsts, out0, out1, acc_sh, ixs, ixd,
                    *bufs):
        rv = bufs[0:_NBUF]
        dv = bufs[_NBUF:2 * _NBUF]
        semr = bufs[2 * _NBUF:3 * _NBUF]
        semd = bufs[3 * _NBUF:4 * _NBUF]
        sems = bufs[4 * _NBUF:5 * _NBUF]
        cid = lax.axis_index("c")
        sid = lax.axis_index("s")
        wid = sid * _NC + cid
        lanes = lax.iota(jnp.int32, _L)
        head_mask = lanes < _HEADS

        pltpu.sync_copy(srcs.at[pl.ds(wid * nchunk, nchunk)], ixs)
        pltpu.sync_copy(dsts.at[pl.ds(wid * nchunk, nchunk)], ixd)

        zero = jnp.zeros((_L,), F32)

        def zrow(i, _):
            for j in range(_REC // _L):
                rv[0][i, pl.ds(j * _L, _L)] = zero
            return 0

        lax.fori_loop(0, _CHUNK, zrow, 0)
        for k, zs in enumerate(zsizes):
            pltpu.sync_copy(rv[0].at[pl.ds(0, zs)],
                            acc_sh.at[pl.ds(sid * rpt + k * _CHUNK, zs)])
        plsc.subcore_barrier()

        def start_gather(c, b):
            pltpu.async_copy(tsrc.at[ixs.at[c]], rv[b], semr[b])
            pltpu.async_copy(tdst.at[ixd.at[c]], dv[b], semd[b])

        def wait_gather(c, b):
            pltpu.make_async_copy(tsrc.at[ixs.at[c]], rv[b], semr[b]).wait()
            pltpu.make_async_copy(tdst.at[ixd.at[c]], dv[b], semd[b]).wait()

        def start_scatter(c, b):
            pltpu.async_copy(rv[b], acc_sh.at[ixd.at[c]], sems[b], add=True)

        def wait_scatter(c, b):
            del c
            pltpu.make_async_copy(tsrc.at[pl.ds(0, _CHUNK)], rv[b],
                                  sems[b]).wait()

        def compute(b):
            rbuf = rv[b]
            dbuf = dv[b]

            def edge(i, _):
                for u in range(4):
                    e = 4 * i + u
                    asv = rbuf[e, pl.ds(_HID, _L)]
                    adv = dbuf[e, pl.ds(0, _L)]
                    t = asv + adv
                    t = jnp.maximum(t, t * 0.2)
                    s = jnp.exp(t)
                    rbuf[e, pl.ds(_HID, _L)] = jnp.where(head_mask, s, 0.0)
                    for k in range(_HEADS):
                        sk = jnp.broadcast_to(s[k], (_L,))
                        c0 = k * _DH
                        rbuf[e, pl.ds(c0, _L)] = rbuf[e, pl.ds(c0, _L)] * sk
                        rbuf[e, pl.ds(c0 + _L, _L)] = \
                            rbuf[e, pl.ds(c0 + _L, _L)] * sk
                return 0

            lax.fori_loop(0, _CHUNK // 4, edge, 0)

        start_gather(0, 0)

        def outer(i, _):
            c0 = i * _NBUF
            for b in range(_NBUF):
                c = c0 + b
                o = 1 - b

                wait_gather(c, b)

                @pl.when(c >= 1)
                def _():
                    wait_scatter(c - 1, o)

                @pl.when(c + 1 < nchunk)
                def _():
                    start_gather(c + 1, o)

                compute(b)
                start_scatter(c, b)
            return 0

        lax.fori_loop(0, nchunk // _NBUF, outer, 0)
        wait_scatter(nchunk - 1, (nchunk - 1) % _NBUF)

        plsc.subcore_barrier()

        @pl.when(cid == 0)
        def _():
            pltpu.sync_copy(acc_sh.at[pl.ds(sid * rpt, rpt)],
                            out0.at[pl.ds(sid * rpt, rpt)])

        @pl.when(cid == 1)
        def _():
            pltpu.sync_copy(acc_sh.at[pl.ds(sid * rpt, rpt)],
                            out1.at[pl.ds(sid * rpt, rpt)])

    return edge_kernel


def kernel(x, edge_index, W_in, b_in, ln1_w, ln1_b, W1, att_src1, att_dst1,
           b1, ln2_w, ln2_b, W2, att_src2, att_dst2, b2, W_out, b_out):
    n = x.shape[0]
    e = edge_index.shape[1]
    r2 = lambda v: v.reshape(1, -1)
    npad = _edge_npad(n)
    bn = npad // 8

    group = _NW * _CHUNK * _NBUF
    e_pad = ((e + group - 1) // group) * group
    nchunk = e_pad // (_NW * _CHUNK)
    pad = jnp.full((2, e_pad - e), n, jnp.int32)
    ei = jnp.concatenate([edge_index, pad], axis=1)
    src = ei[0].reshape(_NW * nchunk, _CHUNK)
    dst = ei[1].reshape(_NW * nchunk, _CHUNK)
    x_p = jnp.concatenate([x, jnp.zeros((npad - n, x.shape[1]), F32)])
    ep = _edge_pass(n, nchunk)

    tsrc1, tdst1 = _stage1(x_p, W_in, r2(b_in), r2(ln1_w), r2(ln1_b), W1,
                           r2(att_src1), r2(att_dst1), bn)
    p0, p1 = ep(tsrc1, tdst1, src, dst)
    tsrc2, tdst2 = _stage2(p0, p1, r2(b1), r2(ln2_w),
                           r2(ln2_b), W2, r2(att_src2), r2(att_dst2), bn)
    q0, q1 = ep(tsrc2, tdst2, src, dst)
    out = _stage3(q0, q1, r2(b2), W_out, b_out[None, :], bn)
    return out[:n]


# speedup vs baseline: 1.5392x; 1.1041x over previous
import functools

import jax
import jax.numpy as jnp
from jax import lax
from jax.experimental import pallas as pl
from jax.experimental.pallas import tpu as pltpu
from jax.experimental.pallas import tpu_sc as plsc

F32 = jnp.float32
_NC = 2
_NS = 16
_L = 16
_NW = _NC * _NS

_HEADS = 4
_DH = 32
_HID = _HEADS * _DH
_REC = 144
_CHUNK = 48

_SQRT_HALF = 0.7071067811865476


def _gelu(x):
    return 0.5 * x * (1.0 + lax.erf(x * _SQRT_HALF))


def _layer_norm(x, w, b):
    mu = jnp.mean(x, axis=-1, keepdims=True)
    var = jnp.mean((x - mu) ** 2, axis=-1, keepdims=True)
    return (x - mu) * lax.rsqrt(var + 1e-5) * w + b


def _attn_mat(att_src_flat, att_dst_flat):
    head = jnp.repeat(jnp.arange(_HEADS), _DH)
    cols = jnp.arange(2 * _HEADS)
    mask = head[:, None] == (cols[None, :] % _HEADS)
    vals = jnp.concatenate([att_src_flat.reshape(-1, 1) *
                            jnp.ones((1, _HEADS), F32),
                            att_dst_flat.reshape(-1, 1) *
                            jnp.ones((1, _HEADS), F32)], axis=1)
    return jnp.where(mask, vals, 0.0)


def _combine(p0, p1, bias):
    acc = p0[:, :_HID] + p1[:, :_HID]
    den = p0[:, _HID:_HID + _HEADS] + p1[:, _HID:_HID + _HEADS]
    outs = []
    for k in range(_HEADS):
        dk = den[:, k:k + 1] + 1e-16
        outs.append(acc[:, _DH * k:_DH * (k + 1)] / dk)
    return jnp.concatenate(outs, axis=1) + bias


def _tables(h, amat):
    bn = h.shape[0]
    att8 = jnp.dot(h, amat, preferred_element_type=F32)
    zsrc = jnp.zeros((bn, _REC - _HID - _HEADS), F32)
    zdst = jnp.zeros((bn, _L - _HEADS), F32)
    tsrc = jnp.concatenate([h, att8[:, :_HEADS], zsrc], axis=1)
    tdst = jnp.concatenate([att8[:, _HEADS:], zdst], axis=1)
    return tsrc, tdst


def _s1_body(x_ref, win_ref, bin_ref, lnw_ref, lnb_ref, w1_ref, am_ref,
             tsrc_ref, tdst_ref):
    x = x_ref[...]
    y = jnp.dot(x, win_ref[...], preferred_element_type=F32) + bin_ref[...]
    y = _gelu(_layer_norm(y, lnw_ref[...], lnb_ref[...]))
    h = jnp.dot(y, w1_ref[...], preferred_element_type=F32)
    tsrc, tdst = _tables(h, am_ref[...])
    tsrc_ref[...] = tsrc
    tdst_ref[...] = tdst


def _s2_body(p0_ref, p1_ref, b1_ref, lnw_ref, lnb_ref, w2_ref, am_ref,
             tsrc_ref, tdst_ref):
    x = _combine(p0_ref[...], p1_ref[...], b1_ref[...])
    y = _gelu(_layer_norm(x, lnw_ref[...], lnb_ref[...]))
    h = jnp.dot(y, w2_ref[...], preferred_element_type=F32)
    tsrc, tdst = _tables(h, am_ref[...])
    tsrc_ref[...] = tsrc
    tdst_ref[...] = tdst


def _s3_body(p0_ref, p1_ref, b2_ref, wout_ref, bout_ref, out_ref):
    x = _combine(p0_ref[...], p1_ref[...], b2_ref[...])
    out_ref[...] = jnp.dot(x, wout_ref[...], preferred_element_type=F32) \
        + bout_ref[...]


def _row_spec(bn, cols):
    return pl.BlockSpec((bn, cols), lambda i: (i, 0))


def _full_spec(shape):
    return pl.BlockSpec(shape, lambda i: (0,) * len(shape))


def _stage1(x, w_in, b_in, ln_w, ln_b, w1, amat, bn):
    n, in_ch = x.shape
    grid = (n // bn,)
    return pl.pallas_call(
        _s1_body,
        grid=grid,
        in_specs=[
            _row_spec(bn, in_ch),
            _full_spec(w_in.shape), _full_spec(b_in.shape),
            _full_spec(ln_w.shape), _full_spec(ln_b.shape),
            _full_spec(w1.shape), _full_spec(amat.shape),
        ],
        out_specs=[_row_spec(bn, _REC), _row_spec(bn, _L)],
        out_shape=[
            jax.ShapeDtypeStruct((n, _REC), F32),
            jax.ShapeDtypeStruct((n, _L), F32),
        ],
    )(x, w_in, b_in, ln_w, ln_b, w1, amat)


def _stage2(p0, p1, b1, ln_w, ln_b, w2, amat, bn):
    n = p0.shape[0]
    grid = (n // bn,)
    return pl.pallas_call(
        _s2_body,
        grid=grid,
        in_specs=[
            _row_spec(bn, _REC), _row_spec(bn, _REC),
            _full_spec(b1.shape),
            _full_spec(ln_w.shape), _full_spec(ln_b.shape),
            _full_spec(w2.shape), _full_spec(amat.shape),
        ],
        out_specs=[_row_spec(bn, _REC), _row_spec(bn, _L)],
        out_shape=[
            jax.ShapeDtypeStruct((n, _REC), F32),
            jax.ShapeDtypeStruct((n, _L), F32),
        ],
    )(p0, p1, b1, ln_w, ln_b, w2, amat)


def _stage3(p0, p1, b2, w_out, b_out, bn):
    n = p0.shape[0]
    out_ch = w_out.shape[1]
    grid = (n // bn,)
    return pl.pallas_call(
        _s3_body,
        grid=grid,
        in_specs=[
            _row_spec(bn, _REC), _row_spec(bn, _REC),
            _full_spec(b2.shape),
            _full_spec(w_out.shape), _full_spec(b_out.shape),
        ],
        out_specs=_row_spec(bn, out_ch),
        out_shape=jax.ShapeDtypeStruct((n, out_ch), F32),
    )(p0, p1, b2, w_out, b_out)


def _edge_npad(n):
    gran = _NS * 8
    return ((n + gran - 1) // gran) * gran


_NBUF = 2


@functools.lru_cache(maxsize=None)
def _edge_pass(n, nchunk):
    assert nchunk % _NBUF == 0
    npad = _edge_npad(n)
    rpt = npad // _NS
    zsizes = [_CHUNK] * (rpt // _CHUNK)
    if rpt % _CHUNK:
        zsizes.append(rpt % _CHUNK)

    mesh = plsc.VectorSubcoreMesh(core_axis_name="c", subcore_axis_name="s",
                                  num_cores=_NC, num_subcores=_NS)

    scratch = [
        pltpu.VMEM_SHARED((npad, _REC), F32),
        pltpu.VMEM((nchunk, _CHUNK), jnp.int32),
        pltpu.VMEM((nchunk, _CHUNK), jnp.int32),
    ]
    scratch += [pltpu.VMEM((_CHUNK, _REC), F32) for _ in range(_NBUF)]
    scratch += [pltpu.VMEM((_CHUNK, _L), F32) for _ in range(_NBUF)]
    scratch += [pltpu.SemaphoreType.DMA for _ in range(3 * _NBUF)]

    @functools.partial(
        pl.kernel,
        out_type=[jax.ShapeDtypeStruct((npad, _REC), F32),
                  jax.ShapeDtypeStruct((npad, _REC), F32)],
        mesh=mesh,
        compiler_params=pltpu.CompilerParams(use_tc_tiling_on_sc=False),
        scratch_types=scratch,
    )
    def edge_kernel(tsrc, tdst, srcs, dsts, out0, out1, acc_sh, ixs, ixd,
                    *bufs):
        rv = bufs[0:_NBUF]
        dv = bufs[_NBUF:2 * _NBUF]
        semr = bufs[2 * _NBUF:3 * _NBUF]
        semd = bufs[3 * _NBUF:4 * _NBUF]
        sems = bufs[4 * _NBUF:5 * _NBUF]
        cid = lax.axis_index("c")
        sid = lax.axis_index("s")
        wid = sid * _NC + cid
        lanes = lax.iota(jnp.int32, _L)
        head_mask = lanes < _HEADS

        pltpu.sync_copy(srcs.at[pl.ds(wid * nchunk, nchunk)], ixs)
        pltpu.sync_copy(dsts.at[pl.ds(wid * nchunk, nchunk)], ixd)

        zero = jnp.zeros((_L,), F32)

        def zrow(i, _):
            for j in range(_REC // _L):
                rv[0][i, pl.ds(j * _L, _L)] = zero
            return 0

        lax.fori_loop(0, _CHUNK, zrow, 0)
        for k, zs in enumerate(zsizes):
            pltpu.sync_copy(rv[0].at[pl.ds(0, zs)],
                            acc_sh.at[pl.ds(sid * rpt + k * _CHUNK, zs)])
        plsc.subcore_barrier()

        def start_gather(c, b):
            pltpu.async_copy(tsrc.at[ixs.at[c]], rv[b], semr[b])
            pltpu.async_copy(tdst.at[ixd.at[c]], dv[b], semd[b])

        def wait_gather(c, b):
            pltpu.make_async_copy(tsrc.at[ixs.at[c]], rv[b], semr[b]).wait()
            pltpu.make_async_copy(tdst.at[ixd.at[c]], dv[b], semd[b]).wait()

        def start_scatter(c, b):
            pltpu.async_copy(rv[b], acc_sh.at[ixd.at[c]], sems[b], add=True)

        def wait_scatter(c, b):
            del c
            pltpu.make_async_copy(tsrc.at[pl.ds(0, _CHUNK)], rv[b],
                                  sems[b]).wait()

        def compute(b):
            rbuf = rv[b]
            dbuf = dv[b]

            def edge(i, _):
                for u in range(4):
                    e = 4 * i + u
                    asv = rbuf[e, pl.ds(_HID, _L)]
                    adv = dbuf[e, pl.ds(0, _L)]
                    t = asv + adv
                    t = jnp.maximum(t, t * 0.2)
                    s = jnp.exp(t)
                    rbuf[e, pl.ds(_HID, _L)] = jnp.where(head_mask, s, 0.0)
                    for k in range(_HEADS):
                        sk = jnp.broadcast_to(s[k], (_L,))
                        c0 = k * _DH
                        rbuf[e, pl.ds(c0, _L)] = rbuf[e, pl.ds(c0, _L)] * sk
                        rbuf[e, pl.ds(c0 + _L, _L)] = \
                            rbuf[e, pl.ds(c0 + _L, _L)] * sk
                return 0

            lax.fori_loop(0, _CHUNK // 4, edge, 0)

        start_gather(0, 0)

        def outer(i, _):
            c0 = i * _NBUF
            for b in range(_NBUF):
                c = c0 + b
                o = 1 - b

                wait_gather(c, b)

                @pl.when(c >= 1)
                def _():
                    wait_scatter(c - 1, o)

                @pl.when(c + 1 < nchunk)
                def _():
                    start_gather(c + 1, o)

                compute(b)
                start_scatter(c, b)
            return 0

        lax.fori_loop(0, nchunk // _NBUF, outer, 0)
        wait_scatter(nchunk - 1, (nchunk - 1) % _NBUF)

        plsc.subcore_barrier()

        @pl.when(cid == 0)
        def _():
            pltpu.sync_copy(acc_sh.at[pl.ds(sid * rpt, rpt)],
                            out0.at[pl.ds(sid * rpt, rpt)])

        @pl.when(cid == 1)
        def _():
            pltpu.sync_copy(acc_sh.at[pl.ds(sid * rpt, rpt)],
                            out1.at[pl.ds(sid * rpt, rpt)])

    return edge_kernel


def kernel(x, edge_index, W_in, b_in, ln1_w, ln1_b, W1, att_src1, att_dst1,
           b1, ln2_w, ln2_b, W2, att_src2, att_dst2, b2, W_out, b_out):
    n = x.shape[0]
    e = edge_index.shape[1]
    r2 = lambda v: v.reshape(1, -1)
    npad = _edge_npad(n)
    bn = npad // 8

    group = _NW * _CHUNK * _NBUF
    e_pad = ((e + group - 1) // group) * group
    nchunk = e_pad // (_NW * _CHUNK)
    pad = jnp.full((2, e_pad - e), n, jnp.int32)
    ei = jnp.concatenate([edge_index, pad], axis=1)
    src = ei[0].reshape(_NW * nchunk, _CHUNK)
    dst = ei[1].reshape(_NW * nchunk, _CHUNK)
    x_p = jnp.concatenate([x, jnp.zeros((npad - n, x.shape[1]), F32)])
    ep = _edge_pass(n, nchunk)

    am1 = _attn_mat(att_src1.reshape(-1), att_dst1.reshape(-1))
    am2 = _attn_mat(att_src2.reshape(-1), att_dst2.reshape(-1))
    tsrc1, tdst1 = _stage1(x_p, W_in, r2(b_in), r2(ln1_w), r2(ln1_b), W1,
                           am1, bn)
    p0, p1 = ep(tsrc1, tdst1, src, dst)
    tsrc2, tdst2 = _stage2(p0, p1, r2(b1), r2(ln2_w),
                           r2(ln2_b), W2, am2, bn)
    q0, q1 = ep(tsrc2, tdst2, src, dst)
    out = _stage3(q0, q1, r2(b2), W_out, b_out[None, :], bn)
    return out[:n]
